# baseline (speedup 1.0000x reference)
import functools

import jax
import jax.numpy as jnp
from jax import lax
from jax.experimental import pallas as pl
from jax.experimental.pallas import tpu as pltpu
from jax.experimental.pallas import tpu_sc as plsc

_LANES = 16
_C = 4
_G = 16


@functools.lru_cache(maxsize=None)
def _build_gmf(batch: int, dim: int):
    info = plsc.get_sparse_core_info()
    nc, ns = info.num_cores, info.num_subcores
    nw = nc * ns
    assert batch % (_G * nw) == 0 and dim == 32
    b_per_w = batch // nw
    n_groups = b_per_w // _G
    out_rows_w = b_per_w * dim // 128

    mesh = plsc.VectorSubcoreMesh(core_axis_name="c", subcore_axis_name="s")

    @functools.partial(
        pl.kernel,
        mesh=mesh,
        compiler_params=pltpu.CompilerParams(needs_layout_passes=False),
        out_type=jax.ShapeDtypeStruct((batch * dim // 128, 128), jnp.float32),
        scratch_types=[
            pltpu.VMEM((b_per_w,), jnp.int32),
            pltpu.VMEM((b_per_w,), jnp.int32),
            pltpu.VMEM((2 * _C, dim, 128), jnp.float32),
            pltpu.VMEM((2 * _C, dim, 128), jnp.float32),
            pltpu.VMEM((out_rows_w, 128), jnp.float32),
            pltpu.SemaphoreType.DMA,
            pltpu.SemaphoreType.DMA,
        ],
    )
    def gmf(uidx_hbm, iidx_hbm, utab_hbm, itab_hbm, out_hbm,
            uidx_v, iidx_v, stage0, stage1, outs, sem0, sem1):
        wid = lax.axis_index("s") * nc + lax.axis_index("c")
        base = wid * b_per_w
        pltpu.sync_copy(uidx_hbm.at[pl.ds(base, b_per_w)], uidx_v)
        pltpu.sync_copy(iidx_hbm.at[pl.ds(base, b_per_w)], iidx_v)

        f_lo = lax.iota(jnp.int32, _LANES)
        stages = [stage0, stage1]
        sems = [sem0, sem1]

        def issue(j, ucol, icol, stage, sem):
            for r in range(_C):
                col = pl.multiple_of(ucol[4 * j + r], 128)
                for t in range(dim // 8):
                    pltpu.async_copy(
                        utab_hbm.at[pl.ds(8 * t, 8), pl.ds(col, 128)],
                        stage.at[r, pl.ds(8 * t, 8)], sem)
            for r in range(_C):
                col = pl.multiple_of(icol[4 * j + r], 128)
                for t in range(dim // 8):
                    pltpu.async_copy(
                        itab_hbm.at[pl.ds(8 * t, 8), pl.ds(col, 128)],
                        stage.at[_C + r, pl.ds(8 * t, 8)], sem)

        def drain(stage, sem):
            for r in range(2 * _C):
                pltpu.make_async_copy(utab_hbm.at[:, pl.ds(0, 128)],
                                      stage.at[r], sem).wait()

        def consume(cbase, j, ulane, ilane, stage):
            for r in range(_C):
                k = 4 * j + r
                urow = jnp.full((_LANES,), r, dtype=jnp.int32)
                irow = jnp.full((_LANES,), _C + r, dtype=jnp.int32)
                ul = jnp.full((_LANES,), ulane[k], dtype=jnp.int32)
                il = jnp.full((_LANES,), ilane[k], dtype=jnp.int32)
                u_lo = plsc.load_gather(stage, [urow, f_lo, ul])
                u_hi = plsc.load_gather(stage, [urow, f_lo + _LANES, ul])
                i_lo = plsc.load_gather(stage, [irow, f_lo, il])
                i_hi = plsc.load_gather(stage, [irow, f_lo + _LANES, il])
                b = cbase + r
                orow = jnp.full((_LANES,), b // 4, dtype=jnp.int32)
                ocol = (b % 4) * 32 + f_lo
                plsc.store_scatter(outs, [orow, ocol], u_lo * i_lo)
                plsc.store_scatter(outs, [orow, ocol + _LANES], u_hi * i_hi)

        def vecs(g):
            uvec = uidx_v[pl.ds(g * _G, _G)]
            ivec = iidx_v[pl.ds(g * _G, _G)]
            return (uvec & jnp.int32(~127), uvec & jnp.int32(127),
                    ivec & jnp.int32(~127), ivec & jnp.int32(127))

        def group(g, carry):
            ulane_p, ilane_p = carry
            ucol, ulane, icol, ilane = vecs(g)
            for j in range(4):
                issue(j, ucol, icol, stages[j & 1], sems[j & 1])
                if j == 0:
                    @pl.when(g > 0)
                    def _():
                        drain(stages[1], sems[1])
                        consume(g * _G - _C, 3, ulane_p, ilane_p, stages[1])
                else:
                    drain(stages[(j - 1) & 1], sems[(j - 1) & 1])
                    consume(g * _G + _C * (j - 1), j - 1, ulane, ilane,
                            stages[(j - 1) & 1])
            return (ulane, ilane)

        zeros = jnp.zeros((_G,), jnp.int32)
        ulane_f, ilane_f = lax.fori_loop(0, n_groups, group, (zeros, zeros))
        drain(stages[1], sems[1])
        consume(b_per_w - _C, 3, ulane_f, ilane_f, stages[1])
        pltpu.sync_copy(outs, out_hbm.at[pl.ds(wid * out_rows_w, out_rows_w)])

    return gmf


def kernel(user_idx, item_idx, user_table, item_table):
    batch, = user_idx.shape
    _, dim = user_table.shape
    gmf = _build_gmf(batch, dim)
    out = gmf(user_idx.astype(jnp.int32), item_idx.astype(jnp.int32),
              user_table.T, item_table.T)
    return out.reshape(batch, dim)


# speedup vs baseline: 1.0111x; 1.0111x over previous
import functools

import jax
import jax.numpy as jnp
from jax import lax
from jax.experimental import pallas as pl
from jax.experimental.pallas import tpu as pltpu
from jax.experimental.pallas import tpu_sc as plsc

_LANES = 16
_C = 4
_G = 16


@functools.lru_cache(maxsize=None)
def _build_gmf(batch: int, dim: int):
    info = plsc.get_sparse_core_info()
    nc, ns = info.num_cores, info.num_subcores
    nw = nc * ns
    assert batch % (_G * nw) == 0 and dim == 32
    b_per_w = batch // nw
    n_groups = b_per_w // _G
    out_rows_w = b_per_w * dim // 128

    mesh = plsc.VectorSubcoreMesh(core_axis_name="c", subcore_axis_name="s")

    @functools.partial(
        pl.kernel,
        mesh=mesh,
        compiler_params=pltpu.CompilerParams(needs_layout_passes=False),
        out_type=jax.ShapeDtypeStruct((batch * dim // 128, 128), jnp.float32),
        scratch_types=[
            pltpu.VMEM((b_per_w,), jnp.int32),
            pltpu.VMEM((b_per_w,), jnp.int32),
            pltpu.VMEM((2 * _C, dim, 128), jnp.float32),
            pltpu.VMEM((2 * _C, dim, 128), jnp.float32),
            pltpu.VMEM((out_rows_w, 128), jnp.float32),
            pltpu.SemaphoreType.DMA,
            pltpu.SemaphoreType.DMA,
        ],
    )
    def gmf(uidx_hbm, iidx_hbm, utab_hbm, itab_hbm, out_hbm,
            uidx_v, iidx_v, stage0, stage1, outs, sem0, sem1):
        wid = lax.axis_index("s") * nc + lax.axis_index("c")
        base = wid * b_per_w
        pltpu.sync_copy(uidx_hbm.at[pl.ds(base, b_per_w)], uidx_v)
        pltpu.sync_copy(iidx_hbm.at[pl.ds(base, b_per_w)], iidx_v)

        f_lo = lax.iota(jnp.int32, _LANES)
        stages = [stage0, stage1]
        sems = [sem0, sem1]

        def issue(j, ucol, icol, stage, sem):
            for r in range(_C):
                col = pl.multiple_of(ucol[4 * j + r], 128)
                pltpu.async_copy(utab_hbm.at[:, pl.ds(col, 128)],
                                 stage.at[r], sem)
            for r in range(_C):
                col = pl.multiple_of(icol[4 * j + r], 128)
                pltpu.async_copy(itab_hbm.at[:, pl.ds(col, 128)],
                                 stage.at[_C + r], sem)

        def drain(stage, sem):
            for r in range(2 * _C):
                pltpu.make_async_copy(utab_hbm.at[:, pl.ds(0, 128)],
                                      stage.at[r], sem).wait()

        def consume(cbase, j, ulane, ilane, stage):
            for r in range(_C):
                k = 4 * j + r
                urow = jnp.full((_LANES,), r, dtype=jnp.int32)
                irow = jnp.full((_LANES,), _C + r, dtype=jnp.int32)
                ul = jnp.full((_LANES,), ulane[k], dtype=jnp.int32)
                il = jnp.full((_LANES,), ilane[k], dtype=jnp.int32)
                u_lo = plsc.load_gather(stage, [urow, f_lo, ul])
                u_hi = plsc.load_gather(stage, [urow, f_lo + _LANES, ul])
                i_lo = plsc.load_gather(stage, [irow, f_lo, il])
                i_hi = plsc.load_gather(stage, [irow, f_lo + _LANES, il])
                b = cbase + r
                orow = jnp.full((_LANES,), b // 4, dtype=jnp.int32)
                ocol = (b % 4) * 32 + f_lo
                plsc.store_scatter(outs, [orow, ocol], u_lo * i_lo)
                plsc.store_scatter(outs, [orow, ocol + _LANES], u_hi * i_hi)

        def vecs(g):
            uvec = uidx_v[pl.ds(g * _G, _G)]
            ivec = iidx_v[pl.ds(g * _G, _G)]
            return (uvec & jnp.int32(~127), uvec & jnp.int32(127),
                    ivec & jnp.int32(~127), ivec & jnp.int32(127))

        def group(g, carry):
            ulane_p, ilane_p = carry
            ucol, ulane, icol, ilane = vecs(g)
            for j in range(4):
                issue(j, ucol, icol, stages[j & 1], sems[j & 1])
                if j == 0:
                    @pl.when(g > 0)
                    def _():
                        drain(stages[1], sems[1])
                        consume(g * _G - _C, 3, ulane_p, ilane_p, stages[1])
                else:
                    drain(stages[(j - 1) & 1], sems[(j - 1) & 1])
                    consume(g * _G + _C * (j - 1), j - 1, ulane, ilane,
                            stages[(j - 1) & 1])
            return (ulane, ilane)

        zeros = jnp.zeros((_G,), jnp.int32)
        ulane_f, ilane_f = lax.fori_loop(0, n_groups, group, (zeros, zeros))
        drain(stages[1], sems[1])
        consume(b_per_w - _C, 3, ulane_f, ilane_f, stages[1])
        pltpu.sync_copy(outs, out_hbm.at[pl.ds(wid * out_rows_w, out_rows_w)])

    return gmf


def kernel(user_idx, item_idx, user_table, item_table):
    batch, = user_idx.shape
    _, dim = user_table.shape
    gmf = _build_gmf(batch, dim)
    out = gmf(user_idx.astype(jnp.int32), item_idx.astype(jnp.int32),
              user_table.T, item_table.T)
    return out.reshape(batch, dim)
